# baseline (speedup 1.0000x reference)
import functools

import jax
import jax.numpy as jnp
from jax.experimental import pallas as pl
from jax.experimental.pallas import tpu as pltpu

_CHUNK = 128


def _upsample2_kernel(x_ref, ew_ref, eh_ref, o_ref, *, n_chunks):
    rows, ws = x_ref.shape[0], ew_ref.shape[1]
    u = jnp.dot(x_ref[...], ew_ref[...], preferred_element_type=o_ref.dtype)
    u3 = u.reshape(rows // 8, 8, ws)
    idx = jax.lax.broadcasted_iota(jnp.int32, (rows // 8, 16, ws), 1) // 2
    o3 = jnp.take_along_axis(u3, idx, axis=1)
    o_ref[...] = o3.reshape(2 * rows, ws)


def _upsample_nearest2(x):
    N, C, H, W = x.shape
    s = 2
    R = N * C * H
    R2 = R * s
    Ws = s * W

    x2 = x.reshape(R, W)

    ew = (jnp.arange(Ws, dtype=jnp.int32)[None, :] // s
          == jnp.arange(W, dtype=jnp.int32)[:, None]).astype(x.dtype)
    eh = (jnp.arange(s * _CHUNK, dtype=jnp.int32)[:, None] // s
          == jnp.arange(_CHUNK, dtype=jnp.int32)[None, :]).astype(x.dtype)

    in_rows = 10240
    n_chunks = in_rows // _CHUNK
    grid = (pl.cdiv(R, in_rows),)

    flops = 2 * R * W * Ws + 2 * R2 * _CHUNK * Ws
    bytes_accessed = (R * W + R2 * Ws) * x.dtype.itemsize

    out2 = pl.pallas_call(
        functools.partial(_upsample2_kernel, n_chunks=n_chunks),
        out_shape=jax.ShapeDtypeStruct((R2, Ws), x.dtype),
        grid=grid,
        in_specs=[
            pl.BlockSpec((in_rows, W), lambda g: (g, 0)),
            pl.BlockSpec((W, Ws), lambda g: (0, 0)),
            pl.BlockSpec((s * _CHUNK, _CHUNK), lambda g: (0, 0)),
        ],
        out_specs=pl.BlockSpec((s * in_rows, Ws), lambda g: (g, 0)),
        compiler_params=pltpu.CompilerParams(
            dimension_semantics=("parallel",),
            vmem_limit_bytes=60 * 1024 * 1024,
        ),
        cost_estimate=pl.CostEstimate(
            flops=flops, transcendentals=0, bytes_accessed=bytes_accessed),
    )(x2, ew, eh)

    return out2.reshape(N, C, s * H, Ws)


def kernel(x):
    return _upsample_nearest2(x)


# speedup vs baseline: 1.0192x; 1.0192x over previous
import jax
import jax.numpy as jnp
from jax.experimental import pallas as pl
from jax.experimental.pallas import tpu as pltpu


def _upsample2_kernel(x_ref, ew_ref, o_ref):
    rows, ws = x_ref.shape[0], ew_ref.shape[1]
    u = jnp.dot(x_ref[...], ew_ref[...], preferred_element_type=o_ref.dtype)
    u3 = u.reshape(rows // 8, 8, ws)
    idx = jax.lax.broadcasted_iota(jnp.int32, (rows // 8, 16, ws), 1) // 2
    o_ref[...] = jnp.take_along_axis(u3, idx, axis=1).reshape(2 * rows, ws)


def _upsample_nearest2(x):
    N, C, H, W = x.shape
    s = 2
    R = N * C * H
    R2 = R * s
    Ws = s * W

    x2 = x.reshape(R, W)

    ew = (jnp.arange(Ws, dtype=jnp.int32)[None, :] // s
          == jnp.arange(W, dtype=jnp.int32)[:, None]).astype(x.dtype)

    in_rows = 8192
    grid = (pl.cdiv(R, in_rows),)

    flops = 2 * R * W * Ws
    bytes_accessed = (R * W + R2 * Ws) * x.dtype.itemsize

    out2 = pl.pallas_call(
        _upsample2_kernel,
        out_shape=jax.ShapeDtypeStruct((R2, Ws), x.dtype),
        grid=grid,
        in_specs=[
            pl.BlockSpec((in_rows, W), lambda g: (g, 0)),
            pl.BlockSpec((W, Ws), lambda g: (0, 0)),
        ],
        out_specs=pl.BlockSpec((s * in_rows, Ws), lambda g: (g, 0)),
        compiler_params=pltpu.CompilerParams(
            dimension_semantics=("parallel",),
            vmem_limit_bytes=48 * 1024 * 1024,
        ),
        cost_estimate=pl.CostEstimate(
            flops=flops, transcendentals=0, bytes_accessed=bytes_accessed),
    )(x2, ew)

    return out2.reshape(N, C, s * H, Ws)


def kernel(x):
    return _upsample_nearest2(x)
